# Initial kernel scaffold; baseline (speedup 1.0000x reference)
import functools

import jax
import jax.numpy as jnp
from jax import lax
from jax.experimental import pallas as pl
from jax.experimental.pallas import tpu as pltpu
from jax.experimental.pallas import tpu_sc as plsc

F32 = jnp.float32

NC, NS = 2, 16
NW = NC * NS
CH = 128
N_R = 10000
NP = 10240
E_R = 320000
EP = 323584
PER_W = EP // NW
N_CHUNK = PER_W // CH
ROWS_PT = NP // NS
DUMMY = N_R
EB = 4096
NB = 1024
N_EB = EP // EB
N_NB = NP // NB

_SC_MESH = dict(core_axis_name="c", subcore_axis_name="s",
                num_cores=NC, num_subcores=NS)


def _wid_base():
    wid = lax.axis_index("s") * NC + lax.axis_index("c")
    return wid * PER_W


def _sc_gather(pr, q, row, col):
    @functools.partial(
        pl.kernel,
        out_type=[jax.ShapeDtypeStruct((EP, 128), F32),
                  jax.ShapeDtypeStruct((EP, 64), F32)],
        mesh=plsc.VectorSubcoreMesh(**_SC_MESH),
        scratch_types=[pltpu.VMEM((CH,), jnp.int32),
                       pltpu.VMEM((CH,), jnp.int32),
                       pltpu.VMEM((CH, 128), F32),
                       pltpu.VMEM((CH, 64), F32),
                       pltpu.SemaphoreType.DMA,
                       pltpu.SemaphoreType.DMA],
    )
    def k(pr_hbm, q_hbm, row_hbm, col_hbm, grow_hbm, gcol_hbm,
          rowv, colv, rbuf, qbuf, sem1, sem2):
        base = _wid_base()

        def body(i, carry):
            off = pl.multiple_of(base + i * CH, 8)
            pltpu.sync_copy(row_hbm.at[pl.ds(off, CH)], rowv)
            pltpu.sync_copy(col_hbm.at[pl.ds(off, CH)], colv)
            c1 = pltpu.async_copy(pr_hbm.at[rowv], rbuf, sem1)
            c2 = pltpu.async_copy(q_hbm.at[colv], qbuf, sem2)
            c1.wait()
            c2.wait()
            pltpu.sync_copy(rbuf, grow_hbm.at[pl.ds(off, CH)])
            pltpu.sync_copy(qbuf, gcol_hbm.at[pl.ds(off, CH)])
            return carry

        lax.fori_loop(0, N_CHUNK, body, 0)

    return k(pr, q, row, col)


def _sc_scatter(m, col, zeros64):
    @functools.partial(
        pl.kernel,
        out_type=jax.ShapeDtypeStruct((NC, NP, 64), F32),
        mesh=plsc.VectorSubcoreMesh(**_SC_MESH),
        scratch_types=[pltpu.VMEM((CH,), jnp.int32),
                       pltpu.VMEM((CH, 64), F32),
                       pltpu.VMEM_SHARED((NP, 64), F32),
                       pltpu.SemaphoreType.DMA],
    )
    def k(m_hbm, col_hbm, z_hbm, out_hbm, colv, mbuf, acc_sh, sem):
        cid = lax.axis_index("c")
        sid = lax.axis_index("s")
        r0 = pl.multiple_of(sid * ROWS_PT, 8)
        pltpu.sync_copy(z_hbm.at[pl.ds(r0, ROWS_PT)],
                        acc_sh.at[pl.ds(r0, ROWS_PT)])
        plsc.subcore_barrier()
        base = _wid_base()

        def body(i, carry):
            off = pl.multiple_of(base + i * CH, 8)
            pltpu.sync_copy(col_hbm.at[pl.ds(off, CH)], colv)
            pltpu.sync_copy(m_hbm.at[pl.ds(off, CH)], mbuf)
            pltpu.sync_copy(mbuf, acc_sh.at[colv], add=True)
            return carry

        lax.fori_loop(0, N_CHUNK, body, 0)
        plsc.subcore_barrier()
        pltpu.sync_copy(acc_sh.at[pl.ds(r0, ROWS_PT)],
                        out_hbm.at[cid, pl.ds(r0, ROWS_PT)])

    return k(m, col, zeros64)


def _sc_count(col, zeros16, ones16):
    @functools.partial(
        pl.kernel,
        out_type=jax.ShapeDtypeStruct((NC, NP, 16), F32),
        mesh=plsc.VectorSubcoreMesh(**_SC_MESH),
        scratch_types=[pltpu.VMEM((CH,), jnp.int32),
                       pltpu.VMEM((CH, 16), F32),
                       pltpu.VMEM_SHARED((NP, 16), F32),
                       pltpu.SemaphoreType.DMA],
    )
    def k(col_hbm, z_hbm, ones_hbm, out_hbm, colv, onesv, acc_sh, sem):
        cid = lax.axis_index("c")
        sid = lax.axis_index("s")
        r0 = pl.multiple_of(sid * ROWS_PT, 8)
        pltpu.sync_copy(z_hbm.at[pl.ds(r0, ROWS_PT)],
                        acc_sh.at[pl.ds(r0, ROWS_PT)])
        pltpu.sync_copy(ones_hbm, onesv)
        plsc.subcore_barrier()
        base = _wid_base()

        def body(i, carry):
            off = pl.multiple_of(base + i * CH, 8)
            pltpu.sync_copy(col_hbm.at[pl.ds(off, CH)], colv)
            pltpu.sync_copy(onesv, acc_sh.at[colv], add=True)
            return carry

        lax.fori_loop(0, N_CHUNK, body, 0)
        plsc.subcore_barrier()
        pltpu.sync_copy(acc_sh.at[pl.ds(r0, ROWS_PT)],
                        out_hbm.at[cid, pl.ds(r0, ROWS_PT)])

    return k(col, zeros16, ones16)


def _full(shape):
    return pl.BlockSpec(shape, lambda i: tuple(0 for _ in shape))


def _rows(shape):
    return pl.BlockSpec(shape, lambda i: (i,) + tuple(0 for _ in shape[1:]))


def _dot(a, b):
    return jax.lax.dot_general(a, b, (((1,), (0,)), ((), ())),
                               preferred_element_type=F32)


def _tc_node_pre(x, batch2d, cond, we, wl1):
    (w1, b1, w2, b2, wc1, bc1, wc2, bc2) = we
    (wea, web, wed, b1e, wma, b1m) = wl1

    def body(x_ref, bt_ref, cond_ref, w1_r, b1_r, w2_r, b2_r,
             wc1_r, bc1_r, wc2_r, bc2_r,
             wea_r, web_r, wed_r, b1e_r, wma_r, b1m_r,
             xh_o, ub_o, pr_o, q_o):
        u = _dot(jax.nn.relu(_dot(cond_ref[...], wc1_r[...]) + bc1_r[...]),
                 wc2_r[...]) + bc2_r[...]
        iota = lax.broadcasted_iota(jnp.int32, (NB, 16), 1)
        oh = (bt_ref[...] == iota).astype(F32)
        ub = _dot(oh, u)
        xh = _dot(jax.nn.relu(_dot(x_ref[...], w1_r[...]) + b1_r[...]),
                  w2_r[...]) + b2_r[...]
        p = _dot(xh, wea_r[...]) + _dot(ub, wed_r[...]) + b1e_r[...]
        r = _dot(xh, wma_r[...]) + b1m_r[...]
        xh_o[...] = xh
        ub_o[...] = ub
        pr_o[...] = jnp.concatenate([p, r], axis=1)
        q_o[...] = _dot(xh, web_r[...])

    return pl.pallas_call(
        body,
        grid=(N_NB,),
        in_specs=[_rows((NB, 128)), _rows((NB, 1)), _full((16, 16)),
                  _full((128, 64)), _full((1, 64)), _full((64, 64)), _full((1, 64)),
                  _full((16, 64)), _full((1, 64)), _full((64, 64)), _full((1, 64)),
                  _full((64, 64)), _full((64, 64)), _full((64, 64)), _full((1, 64)),
                  _full((64, 64)), _full((1, 64))],
        out_specs=[_rows((NB, 64)), _rows((NB, 64)),
                   _rows((NB, 128)), _rows((NB, 64))],
        out_shape=[jax.ShapeDtypeStruct((NP, 64), F32),
                   jax.ShapeDtypeStruct((NP, 64), F32),
                   jax.ShapeDtypeStruct((NP, 128), F32),
                   jax.ShapeDtypeStruct((NP, 64), F32)],
    )(x, batch2d, cond, w1, b1, w2, b2, wc1, bc1, wc2, bc2,
      wea, web, wed, b1e, wma, b1m)


def _tc_edge_enc(ea, w1, b1, w2, b2):
    def body(ea_ref, w1_r, b1_r, w2_r, b2_r, out_ref):
        h = jax.nn.relu(_dot(ea_ref[...], w1_r[...]) + b1_r[...])
        out_ref[...] = _dot(h, w2_r[...]) + b2_r[...]

    return pl.pallas_call(
        body,
        grid=(N_EB,),
        in_specs=[_rows((EB, 16)), _full((16, 64)), _full((1, 64)),
                  _full((64, 64)), _full((1, 64))],
        out_specs=_rows((EB, 64)),
        out_shape=jax.ShapeDtypeStruct((EP, 64), F32),
    )(ea, w1, b1, w2, b2)


def _tc_edge(grow, gcol, eh, wec, w2e, b2e, wmb, w2m, b2m):
    def body(grow_ref, gcol_ref, eh_ref, wec_r, w2e_r, b2e_r,
             wmb_r, w2m_r, b2m_r, ehn_o, m_o):
        g = grow_ref[...]
        h1 = jax.nn.relu(g[:, :64] + gcol_ref[...] + _dot(eh_ref[...], wec_r[...]))
        ehn = _dot(h1, w2e_r[...]) + b2e_r[...]
        h2 = jax.nn.relu(g[:, 64:] + _dot(ehn, wmb_r[...]))
        ehn_o[...] = ehn
        m_o[...] = _dot(h2, w2m_r[...]) + b2m_r[...]

    return pl.pallas_call(
        body,
        grid=(N_EB,),
        in_specs=[_rows((EB, 128)), _rows((EB, 64)), _rows((EB, 64)),
                  _full((64, 64)), _full((64, 64)), _full((1, 64)),
                  _full((64, 64)), _full((64, 64)), _full((1, 64))],
        out_specs=[_rows((EB, 64)), _rows((EB, 64))],
        out_shape=[jax.ShapeDtypeStruct((EP, 64), F32),
                   jax.ShapeDtypeStruct((EP, 64), F32)],
    )(grow, gcol, eh, wec, w2e, b2e, wmb, w2m, b2m)


def _tc_node(xh, s, cnt, ub, wn, wnext):
    (wna, wnb, wnc, b1n, w2n, b2n) = wn
    (wea, web, wed, b1e, wma, b1m) = wnext

    def body(xh_ref, s0_ref, s1_ref, c0_ref, c1_ref, ub_ref,
             wna_r, wnb_r, wnc_r, b1n_r, w2n_r, b2n_r,
             wea_r, web_r, wed_r, b1e_r, wma_r, b1m_r,
             xh_o, pr_o, q_o):
        xh0 = xh_ref[...]
        ub = ub_ref[...]
        cnt = (c0_ref[...] + c1_ref[...])[:, 0:1]
        agg = (s0_ref[...] + s1_ref[...]) / jnp.maximum(cnt, 1.0)
        h = jax.nn.relu(_dot(xh0, wna_r[...]) + _dot(agg, wnb_r[...])
                        + _dot(ub, wnc_r[...]) + b1n_r[...])
        xh = _dot(h, w2n_r[...]) + b2n_r[...] + xh0
        p = _dot(xh, wea_r[...]) + _dot(ub, wed_r[...]) + b1e_r[...]
        r = _dot(xh, wma_r[...]) + b1m_r[...]
        xh_o[...] = xh
        pr_o[...] = jnp.concatenate([p, r], axis=1)
        q_o[...] = _dot(xh, web_r[...])

    return pl.pallas_call(
        body,
        grid=(N_NB,),
        in_specs=[_rows((NB, 64)), _rows((NB, 64)), _rows((NB, 64)),
                  _rows((NB, 16)), _rows((NB, 16)), _rows((NB, 64)),
                  _full((64, 64)), _full((64, 64)), _full((64, 64)),
                  _full((1, 64)), _full((64, 64)), _full((1, 64)),
                  _full((64, 64)), _full((64, 64)), _full((64, 64)),
                  _full((1, 64)), _full((64, 64)), _full((1, 64))],
        out_specs=[_rows((NB, 64)), _rows((NB, 128)), _rows((NB, 64))],
        out_shape=[jax.ShapeDtypeStruct((NP, 64), F32),
                   jax.ShapeDtypeStruct((NP, 128), F32),
                   jax.ShapeDtypeStruct((NP, 64), F32)],
    )(xh, s[0], s[1], cnt[0], cnt[1], ub,
      wna, wnb, wnc, b1n, w2n, b2n, wea, web, wed, b1e, wma, b1m)


def _tc_node_last(xh, s, cnt, ub, wn, wd):
    (wna, wnb, wnc, b1n, w2n, b2n) = wn
    (wd1, bd1, wd2, bd2) = wd

    def body(xh_ref, s0_ref, s1_ref, c0_ref, c1_ref, ub_ref,
             wna_r, wnb_r, wnc_r, b1n_r, w2n_r, b2n_r,
             wd1_r, bd1_r, wd2_r, bd2_r, out_o):
        xh0 = xh_ref[...]
        ub = ub_ref[...]
        cnt = (c0_ref[...] + c1_ref[...])[:, 0:1]
        agg = (s0_ref[...] + s1_ref[...]) / jnp.maximum(cnt, 1.0)
        h = jax.nn.relu(_dot(xh0, wna_r[...]) + _dot(agg, wnb_r[...])
                        + _dot(ub, wnc_r[...]) + b1n_r[...])
        xh = _dot(h, w2n_r[...]) + b2n_r[...] + xh0
        hd = jax.nn.relu(_dot(xh, wd1_r[...]) + bd1_r[...])
        out_o[...] = _dot(hd, wd2_r[...]) + bd2_r[...]

    return pl.pallas_call(
        body,
        grid=(N_NB,),
        in_specs=[_rows((NB, 64)), _rows((NB, 64)), _rows((NB, 64)),
                  _rows((NB, 16)), _rows((NB, 16)), _rows((NB, 64)),
                  _full((64, 64)), _full((64, 64)), _full((64, 64)),
                  _full((1, 64)), _full((64, 64)), _full((1, 64)),
                  _full((64, 64)), _full((1, 64)), _full((64, 128)), _full((1, 128))],
        out_specs=_rows((NB, 128)),
        out_shape=jax.ShapeDtypeStruct((NP, 128), F32),
    )(xh, s[0], s[1], cnt[0], cnt[1], ub,
      wna, wnb, wnc, b1n, w2n, b2n, wd1, bd1, wd2, bd2)


def _row(b):
    return b.reshape(1, -1)


def _split_layer(lp):
    (w1e, b1e), _ = lp["edge_mlp"]
    (w1m, b1m), _ = lp["node_mlp1"]
    return (w1e[0:64], w1e[64:128], w1e[192:256], _row(b1e),
            w1m[0:64], _row(b1m))


def kernel(x, edge_index, edge_attr, conditions, batch, params):
    x = jnp.pad(x, ((0, NP - N_R), (0, 0)))
    batch2d = jnp.pad(batch.astype(jnp.int32), (0, NP - N_R)).reshape(NP, 1)
    row = jnp.pad(edge_index[0].astype(jnp.int32), (0, EP - E_R))
    col = jnp.pad(edge_index[1].astype(jnp.int32), (0, EP - E_R),
                  constant_values=DUMMY)
    ea = jnp.pad(edge_attr, ((0, EP - E_R), (0, 0)))

    zeros64 = jnp.zeros((NP, 64), F32)
    zeros16 = jnp.zeros((NP, 16), F32)
    ones16 = jnp.ones((CH, 16), F32)

    (ne1, ne2) = params["node_enc"]
    (ee1, ee2) = params["edge_enc"]
    (ce1, ce2) = params["cond_enc"]
    enc_w = (ne1[0], _row(ne1[1]), ne2[0], _row(ne2[1]),
             ce1[0], _row(ce1[1]), ce2[0], _row(ce2[1]))

    layers = params["layers"]
    xh, ub, pr, q = _tc_node_pre(x, batch2d, conditions, enc_w,
                                 _split_layer(layers[0]))
    eh = _tc_edge_enc(ea, ee1[0], _row(ee1[1]), ee2[0], _row(ee2[1]))
    cnt = _sc_count(col, zeros16, ones16)

    for li, lp in enumerate(layers):
        (w1e, _), (w2e, b2e) = lp["edge_mlp"]
        (w1m, _), (w2m, b2m) = lp["node_mlp1"]
        grow, gcol = _sc_gather(pr, q, row, col)
        eh, m = _tc_edge(grow, gcol, eh, w1e[128:192], w2e, _row(b2e),
                         w1m[64:128], w2m, _row(b2m))
        s = _sc_scatter(m, col, zeros64)
        (w1n, b1n), (w2n, b2n) = lp["node_mlp2"]
        wn = (w1n[0:64], w1n[64:128], w1n[128:192], _row(b1n), w2n, _row(b2n))
        if li + 1 < len(layers):
            xh, pr, q = _tc_node(xh, s, cnt, ub, wn,
                                 _split_layer(layers[li + 1]))
        else:
            (nd1, nd2) = params["node_dec"]
            out = _tc_node_last(xh, s, cnt, ub, wn,
                                (nd1[0], _row(nd1[1]), nd2[0], _row(nd2[1])))
    return out[:N_R]


# speedup vs baseline: 2.9080x; 2.9080x over previous
import functools

import jax
import jax.numpy as jnp
from jax import lax
from jax.experimental import pallas as pl
from jax.experimental.pallas import tpu as pltpu
from jax.experimental.pallas import tpu_sc as plsc

F32 = jnp.float32

NC, NS = 2, 16
NW = NC * NS
CH = 128
N_R = 10000
NP = 10240
E_R = 320000
EP = 323584
PER_W = EP // NW
N_CHUNK = PER_W // CH
ROWS_PT = NP // NS
DUMMY = N_R
EB = 4096
NB = 1024
N_EB = EP // EB
N_NB = NP // NB

_SC_MESH = dict(core_axis_name="c", subcore_axis_name="s",
                num_cores=NC, num_subcores=NS)


def _wid_base():
    wid = lax.axis_index("s") * NC + lax.axis_index("c")
    return wid * PER_W


def _sc_gather(pr, q, row, col):
    @functools.partial(
        pl.kernel,
        out_type=[jax.ShapeDtypeStruct((EP, 128), F32),
                  jax.ShapeDtypeStruct((EP, 128), F32)],
        mesh=plsc.VectorSubcoreMesh(**_SC_MESH),
        scratch_types=[pltpu.VMEM((CH,), jnp.int32),
                       pltpu.VMEM((CH,), jnp.int32),
                       pltpu.VMEM((CH, 128), F32),
                       pltpu.VMEM((CH, 128), F32),
                       pltpu.SemaphoreType.DMA,
                       pltpu.SemaphoreType.DMA],
    )
    def k(pr_hbm, q_hbm, row_hbm, col_hbm, grow_hbm, gcol_hbm,
          rowv, colv, rbuf, qbuf, sem1, sem2):
        base = _wid_base()

        def body(i, carry):
            off = pl.multiple_of(base + i * CH, 8)
            pltpu.sync_copy(row_hbm.at[pl.ds(off, CH)], rowv)
            pltpu.sync_copy(col_hbm.at[pl.ds(off, CH)], colv)
            c1 = pltpu.async_copy(pr_hbm.at[rowv], rbuf, sem1)
            c2 = pltpu.async_copy(q_hbm.at[colv], qbuf, sem2)
            c1.wait()
            c2.wait()
            pltpu.sync_copy(rbuf, grow_hbm.at[pl.ds(off, CH)])
            pltpu.sync_copy(qbuf, gcol_hbm.at[pl.ds(off, CH)])
            return carry

        lax.fori_loop(0, N_CHUNK, body, 0)

    return k(pr, q, row, col)


def _sc_scatter(m, col, zeros128):
    @functools.partial(
        pl.kernel,
        out_type=jax.ShapeDtypeStruct((NC, NP, 128), F32),
        mesh=plsc.VectorSubcoreMesh(**_SC_MESH),
        scratch_types=[pltpu.VMEM((CH,), jnp.int32),
                       pltpu.VMEM((CH, 128), F32),
                       pltpu.VMEM_SHARED((NP, 128), F32),
                       pltpu.SemaphoreType.DMA],
    )
    def k(m_hbm, col_hbm, z_hbm, out_hbm, colv, mbuf, acc_sh, sem):
        cid = lax.axis_index("c")
        sid = lax.axis_index("s")
        r0 = pl.multiple_of(sid * ROWS_PT, 8)
        pltpu.sync_copy(z_hbm.at[pl.ds(r0, ROWS_PT)],
                        acc_sh.at[pl.ds(r0, ROWS_PT)])
        plsc.subcore_barrier()
        base = _wid_base()

        def body(i, carry):
            off = pl.multiple_of(base + i * CH, 8)
            pltpu.sync_copy(col_hbm.at[pl.ds(off, CH)], colv)
            pltpu.sync_copy(m_hbm.at[pl.ds(off, CH)], mbuf)
            pltpu.sync_copy(mbuf, acc_sh.at[colv], add=True)
            return carry

        lax.fori_loop(0, N_CHUNK, body, 0)
        plsc.subcore_barrier()
        pltpu.sync_copy(acc_sh.at[pl.ds(r0, ROWS_PT)],
                        out_hbm.at[cid, pl.ds(r0, ROWS_PT)])

    return k(m, col, zeros128)


def _sc_count(col, zeros128, ones128):
    @functools.partial(
        pl.kernel,
        out_type=jax.ShapeDtypeStruct((NC, NP, 128), F32),
        mesh=plsc.VectorSubcoreMesh(**_SC_MESH),
        scratch_types=[pltpu.VMEM((CH,), jnp.int32),
                       pltpu.VMEM((CH, 128), F32),
                       pltpu.VMEM_SHARED((NP, 128), F32),
                       pltpu.SemaphoreType.DMA],
    )
    def k(col_hbm, z_hbm, ones_hbm, out_hbm, colv, onesv, acc_sh, sem):
        cid = lax.axis_index("c")
        sid = lax.axis_index("s")
        r0 = pl.multiple_of(sid * ROWS_PT, 8)
        pltpu.sync_copy(z_hbm.at[pl.ds(r0, ROWS_PT)],
                        acc_sh.at[pl.ds(r0, ROWS_PT)])
        pltpu.sync_copy(ones_hbm, onesv)
        plsc.subcore_barrier()
        base = _wid_base()

        def body(i, carry):
            off = pl.multiple_of(base + i * CH, 8)
            pltpu.sync_copy(col_hbm.at[pl.ds(off, CH)], colv)
            pltpu.sync_copy(onesv, acc_sh.at[colv], add=True)
            return carry

        lax.fori_loop(0, N_CHUNK, body, 0)
        plsc.subcore_barrier()
        pltpu.sync_copy(acc_sh.at[pl.ds(r0, ROWS_PT)],
                        out_hbm.at[cid, pl.ds(r0, ROWS_PT)])

    return k(col, zeros128, ones128)


def _full(shape):
    return pl.BlockSpec(shape, lambda i: tuple(0 for _ in shape))


def _rows(shape):
    return pl.BlockSpec(shape, lambda i: (i,) + tuple(0 for _ in shape[1:]))


def _dot(a, b):
    return jax.lax.dot_general(a, b, (((1,), (0,)), ((), ())),
                               preferred_element_type=F32)


def _tc_node_pre(x, batch2d, cond, we, wl1):
    (w1, b1, w2, b2, wc1, bc1, wc2, bc2) = we
    (wea, web, wed, b1e, wma, b1m) = wl1

    def body(x_ref, bt_ref, cond_ref, w1_r, b1_r, w2_r, b2_r,
             wc1_r, bc1_r, wc2_r, bc2_r,
             wea_r, web_r, wed_r, b1e_r, wma_r, b1m_r,
             xh_o, ub_o, pr_o, q_o):
        u = _dot(jax.nn.relu(_dot(cond_ref[...], wc1_r[...]) + bc1_r[...]),
                 wc2_r[...]) + bc2_r[...]
        iota = lax.broadcasted_iota(jnp.int32, (NB, 16), 1)
        oh = (bt_ref[...] == iota).astype(F32)
        ub = _dot(oh, u)
        xh = _dot(jax.nn.relu(_dot(x_ref[...], w1_r[...]) + b1_r[...]),
                  w2_r[...]) + b2_r[...]
        p = _dot(xh, wea_r[...]) + _dot(ub, wed_r[...]) + b1e_r[...]
        r = _dot(xh, wma_r[...]) + b1m_r[...]
        xh_o[...] = xh
        ub_o[...] = ub
        pr_o[...] = jnp.concatenate([p, r], axis=1)
        q_o[...] = jnp.concatenate(
            [_dot(xh, web_r[...]), jnp.zeros((NB, 64), F32)], axis=1)

    return pl.pallas_call(
        body,
        grid=(N_NB,),
        in_specs=[_rows((NB, 128)), _rows((NB, 1)), _full((16, 16)),
                  _full((128, 64)), _full((1, 64)), _full((64, 64)), _full((1, 64)),
                  _full((16, 64)), _full((1, 64)), _full((64, 64)), _full((1, 64)),
                  _full((64, 64)), _full((64, 64)), _full((64, 64)), _full((1, 64)),
                  _full((64, 64)), _full((1, 64))],
        out_specs=[_rows((NB, 64)), _rows((NB, 64)),
                   _rows((NB, 128)), _rows((NB, 128))],
        out_shape=[jax.ShapeDtypeStruct((NP, 64), F32),
                   jax.ShapeDtypeStruct((NP, 64), F32),
                   jax.ShapeDtypeStruct((NP, 128), F32),
                   jax.ShapeDtypeStruct((NP, 128), F32)],
    )(x, batch2d, cond, w1, b1, w2, b2, wc1, bc1, wc2, bc2,
      wea, web, wed, b1e, wma, b1m)


def _tc_edge_enc(ea, w1, b1, w2, b2):
    def body(ea_ref, w1_r, b1_r, w2_r, b2_r, out_ref):
        h = jax.nn.relu(_dot(ea_ref[...], w1_r[...]) + b1_r[...])
        out_ref[...] = _dot(h, w2_r[...]) + b2_r[...]

    return pl.pallas_call(
        body,
        grid=(N_EB,),
        in_specs=[_rows((EB, 16)), _full((16, 64)), _full((1, 64)),
                  _full((64, 64)), _full((1, 64))],
        out_specs=_rows((EB, 64)),
        out_shape=jax.ShapeDtypeStruct((EP, 64), F32),
    )(ea, w1, b1, w2, b2)


def _tc_edge(grow, gcol, eh, wec, w2e, b2e, wmb, w2m, b2m):
    eh_w = eh.shape[1]

    def body(grow_ref, gcol_ref, eh_ref, wec_r, w2e_r, b2e_r,
             wmb_r, w2m_r, b2m_r, em_o):
        g = grow_ref[...]
        ehv = eh_ref[...][:, :64]
        h1 = jax.nn.relu(g[:, :64] + gcol_ref[...][:, :64] + _dot(ehv, wec_r[...]))
        ehn = _dot(h1, w2e_r[...]) + b2e_r[...]
        h2 = jax.nn.relu(g[:, 64:] + _dot(ehn, wmb_r[...]))
        m = _dot(h2, w2m_r[...]) + b2m_r[...]
        em_o[...] = jnp.concatenate([ehn, m], axis=1)

    return pl.pallas_call(
        body,
        grid=(N_EB,),
        in_specs=[_rows((EB, 128)), _rows((EB, 128)),
                  _rows((EB, eh_w)),
                  _full((64, 64)), _full((64, 64)), _full((1, 64)),
                  _full((64, 64)), _full((64, 64)), _full((1, 64))],
        out_specs=_rows((EB, 128)),
        out_shape=jax.ShapeDtypeStruct((EP, 128), F32),
    )(grow, gcol, eh, wec, w2e, b2e, wmb, w2m, b2m)


def _tc_node(xh, s, cnt, ub, wn, wnext):
    (wna, wnb, wnc, b1n, w2n, b2n) = wn
    (wea, web, wed, b1e, wma, b1m) = wnext

    def body(xh_ref, s0_ref, s1_ref, c0_ref, c1_ref, ub_ref,
             wna_r, wnb_r, wnc_r, b1n_r, w2n_r, b2n_r,
             wea_r, web_r, wed_r, b1e_r, wma_r, b1m_r,
             xh_o, pr_o, q_o):
        xh0 = xh_ref[...]
        ub = ub_ref[...]
        cnt = (c0_ref[...] + c1_ref[...])[:, 0:1]
        agg = (s0_ref[...][:, 64:] + s1_ref[...][:, 64:]) / jnp.maximum(cnt, 1.0)
        h = jax.nn.relu(_dot(xh0, wna_r[...]) + _dot(agg, wnb_r[...])
                        + _dot(ub, wnc_r[...]) + b1n_r[...])
        xh = _dot(h, w2n_r[...]) + b2n_r[...] + xh0
        p = _dot(xh, wea_r[...]) + _dot(ub, wed_r[...]) + b1e_r[...]
        r = _dot(xh, wma_r[...]) + b1m_r[...]
        xh_o[...] = xh
        pr_o[...] = jnp.concatenate([p, r], axis=1)
        q_o[...] = jnp.concatenate(
            [_dot(xh, web_r[...]), jnp.zeros((NB, 64), F32)], axis=1)

    s_spec = _rows((NB, 128))
    c_spec = _rows((NB, 128))
    return pl.pallas_call(
        body,
        grid=(N_NB,),
        in_specs=[_rows((NB, 64)), s_spec, s_spec,
                  c_spec, c_spec, _rows((NB, 64)),
                  _full((64, 64)), _full((64, 64)), _full((64, 64)),
                  _full((1, 64)), _full((64, 64)), _full((1, 64)),
                  _full((64, 64)), _full((64, 64)), _full((64, 64)),
                  _full((1, 64)), _full((64, 64)), _full((1, 64))],
        out_specs=[_rows((NB, 64)), _rows((NB, 128)), _rows((NB, 128))],
        out_shape=[jax.ShapeDtypeStruct((NP, 64), F32),
                   jax.ShapeDtypeStruct((NP, 128), F32),
                   jax.ShapeDtypeStruct((NP, 128), F32)],
    )(xh, s[0], s[1], cnt[0], cnt[1], ub,
      wna, wnb, wnc, b1n, w2n, b2n, wea, web, wed, b1e, wma, b1m)


def _tc_node_last(xh, s, cnt, ub, wn, wd):
    (wna, wnb, wnc, b1n, w2n, b2n) = wn
    (wd1, bd1, wd2, bd2) = wd

    def body(xh_ref, s0_ref, s1_ref, c0_ref, c1_ref, ub_ref,
             wna_r, wnb_r, wnc_r, b1n_r, w2n_r, b2n_r,
             wd1_r, bd1_r, wd2_r, bd2_r, out_o):
        xh0 = xh_ref[...]
        ub = ub_ref[...]
        cnt = (c0_ref[...] + c1_ref[...])[:, 0:1]
        agg = (s0_ref[...][:, 64:] + s1_ref[...][:, 64:]) / jnp.maximum(cnt, 1.0)
        h = jax.nn.relu(_dot(xh0, wna_r[...]) + _dot(agg, wnb_r[...])
                        + _dot(ub, wnc_r[...]) + b1n_r[...])
        xh = _dot(h, w2n_r[...]) + b2n_r[...] + xh0
        hd = jax.nn.relu(_dot(xh, wd1_r[...]) + bd1_r[...])
        out_o[...] = _dot(hd, wd2_r[...]) + bd2_r[...]

    s_spec = _rows((NB, 128))
    c_spec = _rows((NB, 128))
    return pl.pallas_call(
        body,
        grid=(N_NB,),
        in_specs=[_rows((NB, 64)), s_spec, s_spec,
                  c_spec, c_spec, _rows((NB, 64)),
                  _full((64, 64)), _full((64, 64)), _full((64, 64)),
                  _full((1, 64)), _full((64, 64)), _full((1, 64)),
                  _full((64, 64)), _full((1, 64)), _full((64, 128)), _full((1, 128))],
        out_specs=_rows((NB, 128)),
        out_shape=jax.ShapeDtypeStruct((NP, 128), F32),
    )(xh, s[0], s[1], cnt[0], cnt[1], ub,
      wna, wnb, wnc, b1n, w2n, b2n, wd1, bd1, wd2, bd2)


def _row(b):
    return b.reshape(1, -1)


def _split_layer(lp):
    (w1e, b1e), _ = lp["edge_mlp"]
    (w1m, b1m), _ = lp["node_mlp1"]
    return (w1e[0:64], w1e[64:128], w1e[192:256], _row(b1e),
            w1m[0:64], _row(b1m))


def kernel(x, edge_index, edge_attr, conditions, batch, params):
    x = jnp.pad(x, ((0, NP - N_R), (0, 0)))
    batch2d = jnp.pad(batch.astype(jnp.int32), (0, NP - N_R)).reshape(NP, 1)
    row = jnp.pad(edge_index[0].astype(jnp.int32), (0, EP - E_R))
    col = jnp.pad(edge_index[1].astype(jnp.int32), (0, EP - E_R),
                  constant_values=DUMMY)
    ea = jnp.pad(edge_attr, ((0, EP - E_R), (0, 0)))

    zeros128 = jnp.zeros((NP, 128), F32)
    ones128 = jnp.ones((CH, 128), F32)

    (ne1, ne2) = params["node_enc"]
    (ee1, ee2) = params["edge_enc"]
    (ce1, ce2) = params["cond_enc"]
    enc_w = (ne1[0], _row(ne1[1]), ne2[0], _row(ne2[1]),
             ce1[0], _row(ce1[1]), ce2[0], _row(ce2[1]))

    layers = params["layers"]
    xh, ub, pr, q = _tc_node_pre(x, batch2d, conditions, enc_w,
                                 _split_layer(layers[0]))
    eh = _tc_edge_enc(ea, ee1[0], _row(ee1[1]), ee2[0], _row(ee2[1]))
    cnt = _sc_count(col, zeros128, ones128)

    for li, lp in enumerate(layers):
        (w1e, _), (w2e, b2e) = lp["edge_mlp"]
        (w1m, _), (w2m, b2m) = lp["node_mlp1"]
        grow, gcol = _sc_gather(pr, q, row, col)
        eh = _tc_edge(grow, gcol, eh, w1e[128:192], w2e, _row(b2e),
                      w1m[64:128], w2m, _row(b2m))
        s = _sc_scatter(eh, col, zeros128)
        (w1n, b1n), (w2n, b2n) = lp["node_mlp2"]
        wn = (w1n[0:64], w1n[64:128], w1n[128:192], _row(b1n), w2n, _row(b2n))
        if li + 1 < len(layers):
            xh, pr, q = _tc_node(xh, s, cnt, ub, wn,
                                 _split_layer(layers[li + 1]))
        else:
            (nd1, nd2) = params["node_dec"]
            out = _tc_node_last(xh, s, cnt, ub, wn,
                                (nd1[0], _row(nd1[1]), nd2[0], _row(nd2[1])))
    return out[:N_R]


# speedup vs baseline: 3.0449x; 1.0471x over previous
import functools

import jax
import jax.numpy as jnp
from jax import lax
from jax.experimental import pallas as pl
from jax.experimental.pallas import tpu as pltpu
from jax.experimental.pallas import tpu_sc as plsc

F32 = jnp.float32

NC, NS = 2, 16
NW = NC * NS
CH = 128
N_R = 10000
NP = 10240
E_R = 320000
EP = 327680
PER_W = EP // NW
N_CHUNK = PER_W // CH
NPAIR = N_CHUNK // 2
ROWS_PT = NP // NS
DUMMY = N_R
EB = 4096
NB = 1024
N_EB = EP // EB
N_NB = NP // NB

_SC_MESH = dict(core_axis_name="c", subcore_axis_name="s",
                num_cores=NC, num_subcores=NS)


def _wid_base():
    wid = lax.axis_index("s") * NC + lax.axis_index("c")
    return wid * PER_W


def _sc_gather(pr, qz, row3, col3):
    @functools.partial(
        pl.kernel,
        out_type=jax.ShapeDtypeStruct((EP, 128), F32),
        mesh=plsc.VectorSubcoreMesh(**_SC_MESH),
        scratch_types=[pltpu.VMEM((N_CHUNK, CH), jnp.int32),
                       pltpu.VMEM((N_CHUNK, CH), jnp.int32),
                       pltpu.VMEM((CH, 128), F32),
                       pltpu.VMEM((CH, 128), F32),
                       pltpu.VMEM((CH, 128), F32),
                       pltpu.VMEM((CH, 128), F32),
                       pltpu.SemaphoreType.DMA,
                       pltpu.SemaphoreType.DMA,
                       pltpu.SemaphoreType.DMA,
                       pltpu.SemaphoreType.DMA],
    )
    def k(pr_hbm, qz_hbm, row3_hbm, col3_hbm, grow_hbm,
          rows_w, cols_w, rbuf0, qbuf0, rbuf1, qbuf1,
          gsem0, gsem1, wsem0, wsem1):
        wid = lax.axis_index("s") * NC + lax.axis_index("c")
        base = wid * PER_W
        pltpu.sync_copy(row3_hbm.at[wid], rows_w)
        pltpu.sync_copy(col3_hbm.at[wid], cols_w)

        def g_issue(j, rbuf, qbuf, gsem):
            pltpu.async_copy(pr_hbm.at[rows_w.at[j]], rbuf, gsem)
            pltpu.async_copy(qz_hbm.at[cols_w.at[j]], qbuf, gsem)

        def g_wait(rbuf, qbuf, gsem):
            pltpu.make_async_copy(pr_hbm.at[rows_w.at[0]], rbuf, gsem).wait()
            pltpu.make_async_copy(qz_hbm.at[cols_w.at[0]], qbuf, gsem).wait()

        def add_q(rbuf, qbuf):
            def rowbody(i, c):
                for t in range(4):
                    sl = pl.ds(t * 16, 16)
                    rbuf[i, sl] = rbuf[i, sl] + qbuf[i, sl]
                return c
            lax.fori_loop(0, CH, rowbody, 0, unroll=8)

        def wb(j, rbuf, wsem):
            off = pl.multiple_of(base + j * CH, 8)
            pltpu.async_copy(rbuf, grow_hbm.at[pl.ds(off, CH)], wsem)

        def wb_wait(rbuf, wsem):
            pltpu.make_async_copy(
                rbuf, grow_hbm.at[pl.ds(pl.multiple_of(base, 8), CH)],
                wsem).wait()

        g_issue(0, rbuf0, qbuf0, gsem0)

        def pair(p, c):
            @pl.when(p > 0)
            def _():
                wb_wait(rbuf1, wsem1)
            g_issue(2 * p + 1, rbuf1, qbuf1, gsem1)
            g_wait(rbuf0, qbuf0, gsem0)
            add_q(rbuf0, qbuf0)
            wb(2 * p, rbuf0, wsem0)

            @pl.when(p + 1 < NPAIR)
            def _():
                wb_wait(rbuf0, wsem0)
                g_issue(2 * p + 2, rbuf0, qbuf0, gsem0)
            g_wait(rbuf1, qbuf1, gsem1)
            add_q(rbuf1, qbuf1)
            wb(2 * p + 1, rbuf1, wsem1)
            return c

        lax.fori_loop(0, NPAIR, pair, 0)
        wb_wait(rbuf0, wsem0)
        wb_wait(rbuf1, wsem1)

    return k(pr, qz, row3, col3)


def _sc_scatter(em, col3, zeros128):
    @functools.partial(
        pl.kernel,
        out_type=jax.ShapeDtypeStruct((NC, NP, 128), F32),
        mesh=plsc.VectorSubcoreMesh(**_SC_MESH),
        scratch_types=[pltpu.VMEM((N_CHUNK, CH), jnp.int32),
                       pltpu.VMEM((CH, 128), F32),
                       pltpu.VMEM((CH, 128), F32),
                       pltpu.VMEM_SHARED((NP, 128), F32),
                       pltpu.SemaphoreType.DMA,
                       pltpu.SemaphoreType.DMA,
                       pltpu.SemaphoreType.DMA,
                       pltpu.SemaphoreType.DMA],
    )
    def k(em_hbm, col3_hbm, z_hbm, out_hbm,
          cols_w, mbuf0, mbuf1, acc_sh, lsem0, lsem1, ssem0, ssem1):
        cid = lax.axis_index("c")
        sid = lax.axis_index("s")
        wid = sid * NC + cid
        base = wid * PER_W
        r0 = pl.multiple_of(sid * ROWS_PT, 8)
        pltpu.sync_copy(z_hbm.at[pl.ds(r0, ROWS_PT)],
                        acc_sh.at[pl.ds(r0, ROWS_PT)])
        pltpu.sync_copy(col3_hbm.at[wid], cols_w)
        plsc.subcore_barrier()

        def m_issue(j, mbuf, lsem):
            off = pl.multiple_of(base + j * CH, 8)
            pltpu.async_copy(em_hbm.at[pl.ds(off, CH)], mbuf, lsem)

        def m_wait(mbuf, lsem):
            pltpu.make_async_copy(
                em_hbm.at[pl.ds(pl.multiple_of(base, 8), CH)], mbuf,
                lsem).wait()

        def s_issue(j, mbuf, ssem):
            pltpu.async_copy(mbuf, acc_sh.at[cols_w.at[j]], ssem, add=True)

        def s_wait(mbuf, ssem):
            pltpu.make_async_copy(mbuf, acc_sh.at[cols_w.at[0]], ssem).wait()

        m_issue(0, mbuf0, lsem0)

        def pair(p, c):
            @pl.when(p > 0)
            def _():
                s_wait(mbuf1, ssem1)
            m_issue(2 * p + 1, mbuf1, lsem1)
            m_wait(mbuf0, lsem0)
            s_issue(2 * p, mbuf0, ssem0)

            @pl.when(p + 1 < NPAIR)
            def _():
                s_wait(mbuf0, ssem0)
                m_issue(2 * p + 2, mbuf0, lsem0)
            m_wait(mbuf1, lsem1)
            s_issue(2 * p + 1, mbuf1, ssem1)
            return c

        lax.fori_loop(0, NPAIR, pair, 0)
        s_wait(mbuf0, ssem0)
        s_wait(mbuf1, ssem1)
        plsc.subcore_barrier()
        pltpu.sync_copy(acc_sh.at[pl.ds(r0, ROWS_PT)],
                        out_hbm.at[cid, pl.ds(r0, ROWS_PT)])

    return k(em, col3, zeros128)


def _sc_count(col3, zeros128, ones128):
    @functools.partial(
        pl.kernel,
        out_type=jax.ShapeDtypeStruct((NC, NP, 128), F32),
        mesh=plsc.VectorSubcoreMesh(**_SC_MESH),
        scratch_types=[pltpu.VMEM((N_CHUNK, CH), jnp.int32),
                       pltpu.VMEM((CH, 128), F32),
                       pltpu.VMEM_SHARED((NP, 128), F32),
                       pltpu.SemaphoreType.DMA,
                       pltpu.SemaphoreType.DMA],
    )
    def k(col3_hbm, z_hbm, ones_hbm, out_hbm,
          cols_w, onesv, acc_sh, ssem0, ssem1):
        cid = lax.axis_index("c")
        sid = lax.axis_index("s")
        wid = sid * NC + cid
        r0 = pl.multiple_of(sid * ROWS_PT, 8)
        pltpu.sync_copy(z_hbm.at[pl.ds(r0, ROWS_PT)],
                        acc_sh.at[pl.ds(r0, ROWS_PT)])
        pltpu.sync_copy(ones_hbm, onesv)
        pltpu.sync_copy(col3_hbm.at[wid], cols_w)
        plsc.subcore_barrier()

        def s_issue(j, ssem):
            pltpu.async_copy(onesv, acc_sh.at[cols_w.at[j]], ssem, add=True)

        def s_wait(ssem):
            pltpu.make_async_copy(onesv, acc_sh.at[cols_w.at[0]], ssem).wait()

        s_issue(0, ssem0)

        def pair(p, c):
            @pl.when(p > 0)
            def _():
                s_wait(ssem1)
            s_issue(2 * p + 1, ssem1)
            s_wait(ssem0)

            @pl.when(p + 1 < NPAIR)
            def _():
                s_issue(2 * p + 2, ssem0)
            return c

        lax.fori_loop(0, NPAIR, pair, 0)
        s_wait(ssem1)
        plsc.subcore_barrier()
        pltpu.sync_copy(acc_sh.at[pl.ds(r0, ROWS_PT)],
                        out_hbm.at[cid, pl.ds(r0, ROWS_PT)])

    return k(col3, zeros128, ones128)


def _full(shape):
    return pl.BlockSpec(shape, lambda i: tuple(0 for _ in shape))


def _rows(shape):
    return pl.BlockSpec(shape, lambda i: (i,) + tuple(0 for _ in shape[1:]))


def _dot(a, b):
    return jax.lax.dot_general(a, b, (((1,), (0,)), ((), ())),
                               preferred_element_type=F32)


def _tc_node_pre(x, batch2d, cond, we, wl1):
    (w1, b1, w2, b2, wc1, bc1, wc2, bc2) = we
    (wea, web, wed, b1e, wma, b1m) = wl1

    def body(x_ref, bt_ref, cond_ref, w1_r, b1_r, w2_r, b2_r,
             wc1_r, bc1_r, wc2_r, bc2_r,
             wea_r, web_r, wed_r, b1e_r, wma_r, b1m_r,
             xh_o, ub_o, pr_o, q_o):
        u = _dot(jax.nn.relu(_dot(cond_ref[...], wc1_r[...]) + bc1_r[...]),
                 wc2_r[...]) + bc2_r[...]
        iota = lax.broadcasted_iota(jnp.int32, (NB, 16), 1)
        oh = (bt_ref[...] == iota).astype(F32)
        ub = _dot(oh, u)
        xh = _dot(jax.nn.relu(_dot(x_ref[...], w1_r[...]) + b1_r[...]),
                  w2_r[...]) + b2_r[...]
        p = _dot(xh, wea_r[...]) + _dot(ub, wed_r[...]) + b1e_r[...]
        r = _dot(xh, wma_r[...]) + b1m_r[...]
        xh_o[...] = xh
        ub_o[...] = ub
        pr_o[...] = jnp.concatenate([p, r], axis=1)
        q_o[...] = jnp.concatenate(
            [_dot(xh, web_r[...]), jnp.zeros((NB, 64), F32)], axis=1)

    return pl.pallas_call(
        body,
        grid=(N_NB,),
        in_specs=[_rows((NB, 128)), _rows((NB, 1)), _full((16, 16)),
                  _full((128, 64)), _full((1, 64)), _full((64, 64)), _full((1, 64)),
                  _full((16, 64)), _full((1, 64)), _full((64, 64)), _full((1, 64)),
                  _full((64, 64)), _full((64, 64)), _full((64, 64)), _full((1, 64)),
                  _full((64, 64)), _full((1, 64))],
        out_specs=[_rows((NB, 64)), _rows((NB, 64)),
                   _rows((NB, 128)), _rows((NB, 128))],
        out_shape=[jax.ShapeDtypeStruct((NP, 64), F32),
                   jax.ShapeDtypeStruct((NP, 64), F32),
                   jax.ShapeDtypeStruct((NP, 128), F32),
                   jax.ShapeDtypeStruct((NP, 128), F32)],
    )(x, batch2d, cond, w1, b1, w2, b2, wc1, bc1, wc2, bc2,
      wea, web, wed, b1e, wma, b1m)


def _tc_edge_enc(ea, w1, b1, w2, b2):
    def body(ea_ref, w1_r, b1_r, w2_r, b2_r, out_ref):
        h = jax.nn.relu(_dot(ea_ref[...], w1_r[...]) + b1_r[...])
        out_ref[...] = _dot(h, w2_r[...]) + b2_r[...]

    return pl.pallas_call(
        body,
        grid=(N_EB,),
        in_specs=[_rows((EB, 16)), _full((16, 64)), _full((1, 64)),
                  _full((64, 64)), _full((1, 64))],
        out_specs=_rows((EB, 64)),
        out_shape=jax.ShapeDtypeStruct((EP, 64), F32),
    )(ea, w1, b1, w2, b2)


def _tc_edge(grow, eh, wec, w2e, b2e, wmb, w2m, b2m):
    eh_w = eh.shape[1]

    def body(grow_ref, eh_ref, wec_r, w2e_r, b2e_r,
             wmb_r, w2m_r, b2m_r, em_o):
        g = grow_ref[...]
        ehv = eh_ref[...][:, :64]
        h1 = jax.nn.relu(g[:, :64] + _dot(ehv, wec_r[...]))
        ehn = _dot(h1, w2e_r[...]) + b2e_r[...]
        h2 = jax.nn.relu(g[:, 64:] + _dot(ehn, wmb_r[...]))
        m = _dot(h2, w2m_r[...]) + b2m_r[...]
        em_o[...] = jnp.concatenate([ehn, m], axis=1)

    return pl.pallas_call(
        body,
        grid=(N_EB,),
        in_specs=[_rows((EB, 128)),
                  _rows((EB, eh_w)),
                  _full((64, 64)), _full((64, 64)), _full((1, 64)),
                  _full((64, 64)), _full((64, 64)), _full((1, 64))],
        out_specs=_rows((EB, 128)),
        out_shape=jax.ShapeDtypeStruct((EP, 128), F32),
    )(grow, eh, wec, w2e, b2e, wmb, w2m, b2m)


def _tc_node(xh, s, cnt, ub, wn, wnext):
    (wna, wnb, wnc, b1n, w2n, b2n) = wn
    (wea, web, wed, b1e, wma, b1m) = wnext

    def body(xh_ref, s0_ref, s1_ref, c0_ref, c1_ref, ub_ref,
             wna_r, wnb_r, wnc_r, b1n_r, w2n_r, b2n_r,
             wea_r, web_r, wed_r, b1e_r, wma_r, b1m_r,
             xh_o, pr_o, q_o):
        xh0 = xh_ref[...]
        ub = ub_ref[...]
        cnt = (c0_ref[...] + c1_ref[...])[:, 0:1]
        agg = (s0_ref[...][:, 64:] + s1_ref[...][:, 64:]) / jnp.maximum(cnt, 1.0)
        h = jax.nn.relu(_dot(xh0, wna_r[...]) + _dot(agg, wnb_r[...])
                        + _dot(ub, wnc_r[...]) + b1n_r[...])
        xh = _dot(h, w2n_r[...]) + b2n_r[...] + xh0
        p = _dot(xh, wea_r[...]) + _dot(ub, wed_r[...]) + b1e_r[...]
        r = _dot(xh, wma_r[...]) + b1m_r[...]
        xh_o[...] = xh
        pr_o[...] = jnp.concatenate([p, r], axis=1)
        q_o[...] = jnp.concatenate(
            [_dot(xh, web_r[...]), jnp.zeros((NB, 64), F32)], axis=1)

    s_spec = _rows((NB, 128))
    c_spec = _rows((NB, 128))
    return pl.pallas_call(
        body,
        grid=(N_NB,),
        in_specs=[_rows((NB, 64)), s_spec, s_spec,
                  c_spec, c_spec, _rows((NB, 64)),
                  _full((64, 64)), _full((64, 64)), _full((64, 64)),
                  _full((1, 64)), _full((64, 64)), _full((1, 64)),
                  _full((64, 64)), _full((64, 64)), _full((64, 64)),
                  _full((1, 64)), _full((64, 64)), _full((1, 64))],
        out_specs=[_rows((NB, 64)), _rows((NB, 128)), _rows((NB, 128))],
        out_shape=[jax.ShapeDtypeStruct((NP, 64), F32),
                   jax.ShapeDtypeStruct((NP, 128), F32),
                   jax.ShapeDtypeStruct((NP, 128), F32)],
    )(xh, s[0], s[1], cnt[0], cnt[1], ub,
      wna, wnb, wnc, b1n, w2n, b2n, wea, web, wed, b1e, wma, b1m)


def _tc_node_last(xh, s, cnt, ub, wn, wd):
    (wna, wnb, wnc, b1n, w2n, b2n) = wn
    (wd1, bd1, wd2, bd2) = wd

    def body(xh_ref, s0_ref, s1_ref, c0_ref, c1_ref, ub_ref,
             wna_r, wnb_r, wnc_r, b1n_r, w2n_r, b2n_r,
             wd1_r, bd1_r, wd2_r, bd2_r, out_o):
        xh0 = xh_ref[...]
        ub = ub_ref[...]
        cnt = (c0_ref[...] + c1_ref[...])[:, 0:1]
        agg = (s0_ref[...][:, 64:] + s1_ref[...][:, 64:]) / jnp.maximum(cnt, 1.0)
        h = jax.nn.relu(_dot(xh0, wna_r[...]) + _dot(agg, wnb_r[...])
                        + _dot(ub, wnc_r[...]) + b1n_r[...])
        xh = _dot(h, w2n_r[...]) + b2n_r[...] + xh0
        hd = jax.nn.relu(_dot(xh, wd1_r[...]) + bd1_r[...])
        out_o[...] = _dot(hd, wd2_r[...]) + bd2_r[...]

    s_spec = _rows((NB, 128))
    c_spec = _rows((NB, 128))
    return pl.pallas_call(
        body,
        grid=(N_NB,),
        in_specs=[_rows((NB, 64)), s_spec, s_spec,
                  c_spec, c_spec, _rows((NB, 64)),
                  _full((64, 64)), _full((64, 64)), _full((64, 64)),
                  _full((1, 64)), _full((64, 64)), _full((1, 64)),
                  _full((64, 64)), _full((1, 64)), _full((64, 128)), _full((1, 128))],
        out_specs=_rows((NB, 128)),
        out_shape=jax.ShapeDtypeStruct((NP, 128), F32),
    )(xh, s[0], s[1], cnt[0], cnt[1], ub,
      wna, wnb, wnc, b1n, w2n, b2n, wd1, bd1, wd2, bd2)


def _row(b):
    return b.reshape(1, -1)


def _split_layer(lp):
    (w1e, b1e), _ = lp["edge_mlp"]
    (w1m, b1m), _ = lp["node_mlp1"]
    return (w1e[0:64], w1e[64:128], w1e[192:256], _row(b1e),
            w1m[0:64], _row(b1m))


def kernel(x, edge_index, edge_attr, conditions, batch, params):
    x = jnp.pad(x, ((0, NP - N_R), (0, 0)))
    batch2d = jnp.pad(batch.astype(jnp.int32), (0, NP - N_R)).reshape(NP, 1)
    row3 = jnp.pad(edge_index[0].astype(jnp.int32),
                   (0, EP - E_R)).reshape(NW, N_CHUNK, CH)
    col3 = jnp.pad(edge_index[1].astype(jnp.int32), (0, EP - E_R),
                   constant_values=DUMMY).reshape(NW, N_CHUNK, CH)
    ea = jnp.pad(edge_attr, ((0, EP - E_R), (0, 0)))

    zeros128 = jnp.zeros((NP, 128), F32)
    ones128 = jnp.ones((CH, 128), F32)

    (ne1, ne2) = params["node_enc"]
    (ee1, ee2) = params["edge_enc"]
    (ce1, ce2) = params["cond_enc"]
    enc_w = (ne1[0], _row(ne1[1]), ne2[0], _row(ne2[1]),
             ce1[0], _row(ce1[1]), ce2[0], _row(ce2[1]))

    layers = params["layers"]
    xh, ub, pr, q = _tc_node_pre(x, batch2d, conditions, enc_w,
                                 _split_layer(layers[0]))
    eh = _tc_edge_enc(ea, ee1[0], _row(ee1[1]), ee2[0], _row(ee2[1]))
    cnt = _sc_count(col3, zeros128, ones128)

    for li, lp in enumerate(layers):
        (w1e, _), (w2e, b2e) = lp["edge_mlp"]
        (w1m, _), (w2m, b2m) = lp["node_mlp1"]
        grow = _sc_gather(pr, q, row3, col3)
        eh = _tc_edge(grow, eh, w1e[128:192], w2e, _row(b2e),
                      w1m[64:128], w2m, _row(b2m))
        s = _sc_scatter(eh, col3, zeros128)
        (w1n, b1n), (w2n, b2n) = lp["node_mlp2"]
        wn = (w1n[0:64], w1n[64:128], w1n[128:192], _row(b1n), w2n, _row(b2n))
        if li + 1 < len(layers):
            xh, pr, q = _tc_node(xh, s, cnt, ub, wn,
                                 _split_layer(layers[li + 1]))
        else:
            (nd1, nd2) = params["node_dec"]
            out = _tc_node_last(xh, s, cnt, ub, wn,
                                (nd1[0], _row(nd1[1]), nd2[0], _row(nd2[1])))
    return out[:N_R]


# speedup vs baseline: 3.5063x; 1.1515x over previous
import functools

import jax
import jax.numpy as jnp
from jax import lax
from jax.experimental import pallas as pl
from jax.experimental.pallas import tpu as pltpu
from jax.experimental.pallas import tpu_sc as plsc

F32 = jnp.float32

NC, NS = 2, 16
NW = NC * NS
CH = 128
N_R = 10000
NP = 10240
E_R = 320000
EP = 327680
PER_W = EP // NW
N_CHUNK = PER_W // CH
NPAIR = N_CHUNK // 2
ROWS_PT = NP // NS
DUMMY = N_R
EB = 4096
NB = 1024
N_EB = EP // EB
N_NB = NP // NB

_SC_MESH = dict(core_axis_name="c", subcore_axis_name="s",
                num_cores=NC, num_subcores=NS)


def _wid_base():
    wid = lax.axis_index("s") * NC + lax.axis_index("c")
    return wid * PER_W


def _sc_gather(pr, qz, row3, col3):
    n_chunk = row3.shape[1]
    per_w = n_chunk * CH
    ep = NW * per_w
    nbuf = 4
    ngrp = n_chunk // nbuf

    @functools.partial(
        pl.kernel,
        out_type=jax.ShapeDtypeStruct((ep, 128), F32),
        mesh=plsc.VectorSubcoreMesh(**_SC_MESH),
        compiler_params=pltpu.CompilerParams(use_tc_tiling_on_sc=False),
        scratch_types=[pltpu.VMEM((n_chunk, CH), jnp.int32),
                       pltpu.VMEM((n_chunk, CH), jnp.int32)]
                      + [pltpu.VMEM((CH, 128), F32) for _ in range(nbuf)]
                      + [pltpu.VMEM((CH, 64), F32) for _ in range(nbuf)]
                      + [pltpu.SemaphoreType.DMA for _ in range(2 * nbuf)],
    )
    def k(pr_hbm, qz_hbm, row3_hbm, col3_hbm, grow_hbm, rows_w, cols_w, *bufs):
        rbufs = bufs[0:nbuf]
        qbufs = bufs[nbuf:2 * nbuf]
        gsems = bufs[2 * nbuf:3 * nbuf]
        wsems = bufs[3 * nbuf:4 * nbuf]
        wid = lax.axis_index("s") * NC + lax.axis_index("c")
        base = wid * per_w
        pltpu.sync_copy(row3_hbm.at[wid], rows_w)
        pltpu.sync_copy(col3_hbm.at[wid], cols_w)

        def g_issue(j, b):
            pltpu.async_copy(pr_hbm.at[rows_w.at[j]], rbufs[b], gsems[b])
            pltpu.async_copy(qz_hbm.at[cols_w.at[j]], qbufs[b], gsems[b])

        def g_wait(b):
            pltpu.make_async_copy(pr_hbm.at[rows_w.at[0]], rbufs[b],
                                  gsems[b]).wait()
            pltpu.make_async_copy(qz_hbm.at[cols_w.at[0]], qbufs[b],
                                  gsems[b]).wait()

        def add_q(b):
            def rowbody(i, c):
                for t in range(4):
                    sl = pl.ds(t * 16, 16)
                    rbufs[b][i, sl] = rbufs[b][i, sl] + qbufs[b][i, sl]
                return c
            lax.fori_loop(0, CH, rowbody, 0, unroll=8)

        def wb(j, b):
            off = pl.multiple_of(base + j * CH, 8)
            pltpu.async_copy(rbufs[b], grow_hbm.at[pl.ds(off, CH)], wsems[b])

        def wb_wait(b):
            pltpu.make_async_copy(
                rbufs[b], grow_hbm.at[pl.ds(pl.multiple_of(base, 8), CH)],
                wsems[b]).wait()

        for b in range(nbuf):
            g_issue(b, b)

        def group(p, c):
            for b in range(nbuf):
                j = nbuf * p + b
                g_wait(b)
                add_q(b)
                wb(j, b)

                @pl.when(j + nbuf < n_chunk)
                def _():
                    wb_wait(b)
                    g_issue(j + nbuf, b)
            return c

        lax.fori_loop(0, ngrp, group, 0)
        for b in range(nbuf):
            wb_wait(b)

    return k(pr, qz, row3, col3)


def _sc_scatter(em, col3, init):
    n_chunk = col3.shape[1]
    npair = n_chunk // 2
    per_w = n_chunk * CH
    per_core_init = init.ndim == 3

    @functools.partial(
        pl.kernel,
        out_type=jax.ShapeDtypeStruct((NC, NP, 128), F32),
        mesh=plsc.VectorSubcoreMesh(**_SC_MESH),
        scratch_types=[pltpu.VMEM((n_chunk, CH), jnp.int32),
                       pltpu.VMEM((CH, 128), F32),
                       pltpu.VMEM((CH, 128), F32),
                       pltpu.VMEM_SHARED((NP, 128), F32),
                       pltpu.SemaphoreType.DMA,
                       pltpu.SemaphoreType.DMA,
                       pltpu.SemaphoreType.DMA,
                       pltpu.SemaphoreType.DMA],
    )
    def k(em_hbm, col3_hbm, z_hbm, out_hbm,
          cols_w, mbuf0, mbuf1, acc_sh, lsem0, lsem1, ssem0, ssem1):
        cid = lax.axis_index("c")
        sid = lax.axis_index("s")
        wid = sid * NC + cid
        base = wid * per_w
        r0 = pl.multiple_of(sid * ROWS_PT, 8)
        if per_core_init:
            pltpu.sync_copy(z_hbm.at[cid, pl.ds(r0, ROWS_PT)],
                            acc_sh.at[pl.ds(r0, ROWS_PT)])
        else:
            pltpu.sync_copy(z_hbm.at[pl.ds(r0, ROWS_PT)],
                            acc_sh.at[pl.ds(r0, ROWS_PT)])
        pltpu.sync_copy(col3_hbm.at[wid], cols_w)
        plsc.subcore_barrier()

        def m_issue(j, mbuf, lsem):
            off = pl.multiple_of(base + j * CH, 8)
            pltpu.async_copy(em_hbm.at[pl.ds(off, CH)], mbuf, lsem)

        def m_wait(mbuf, lsem):
            pltpu.make_async_copy(
                em_hbm.at[pl.ds(pl.multiple_of(base, 8), CH)], mbuf,
                lsem).wait()

        def s_issue(j, mbuf, ssem):
            pltpu.async_copy(mbuf, acc_sh.at[cols_w.at[j]], ssem, add=True)

        def s_wait(mbuf, ssem):
            pltpu.make_async_copy(mbuf, acc_sh.at[cols_w.at[0]], ssem).wait()

        m_issue(0, mbuf0, lsem0)

        def pair(p, c):
            @pl.when(p > 0)
            def _():
                s_wait(mbuf1, ssem1)
            m_issue(2 * p + 1, mbuf1, lsem1)
            m_wait(mbuf0, lsem0)
            s_issue(2 * p, mbuf0, ssem0)

            @pl.when(p + 1 < npair)
            def _():
                s_wait(mbuf0, ssem0)
                m_issue(2 * p + 2, mbuf0, lsem0)
            m_wait(mbuf1, lsem1)
            s_issue(2 * p + 1, mbuf1, ssem1)
            return c

        lax.fori_loop(0, npair, pair, 0)
        s_wait(mbuf0, ssem0)
        s_wait(mbuf1, ssem1)
        plsc.subcore_barrier()
        pltpu.sync_copy(acc_sh.at[pl.ds(r0, ROWS_PT)],
                        out_hbm.at[cid, pl.ds(r0, ROWS_PT)])

    return k(em, col3, init)


def _sc_count(col3, zeros128, ones128):
    @functools.partial(
        pl.kernel,
        out_type=jax.ShapeDtypeStruct((NC, NP, 128), F32),
        mesh=plsc.VectorSubcoreMesh(**_SC_MESH),
        scratch_types=[pltpu.VMEM((N_CHUNK, CH), jnp.int32),
                       pltpu.VMEM((CH, 128), F32),
                       pltpu.VMEM_SHARED((NP, 128), F32),
                       pltpu.SemaphoreType.DMA,
                       pltpu.SemaphoreType.DMA],
    )
    def k(col3_hbm, z_hbm, ones_hbm, out_hbm,
          cols_w, onesv, acc_sh, ssem0, ssem1):
        cid = lax.axis_index("c")
        sid = lax.axis_index("s")
        wid = sid * NC + cid
        r0 = pl.multiple_of(sid * ROWS_PT, 8)
        pltpu.sync_copy(z_hbm.at[pl.ds(r0, ROWS_PT)],
                        acc_sh.at[pl.ds(r0, ROWS_PT)])
        pltpu.sync_copy(ones_hbm, onesv)
        pltpu.sync_copy(col3_hbm.at[wid], cols_w)
        plsc.subcore_barrier()

        def s_issue(j, ssem):
            pltpu.async_copy(onesv, acc_sh.at[cols_w.at[j]], ssem, add=True)

        def s_wait(ssem):
            pltpu.make_async_copy(onesv, acc_sh.at[cols_w.at[0]], ssem).wait()

        s_issue(0, ssem0)

        def pair(p, c):
            @pl.when(p > 0)
            def _():
                s_wait(ssem1)
            s_issue(2 * p + 1, ssem1)
            s_wait(ssem0)

            @pl.when(p + 1 < NPAIR)
            def _():
                s_issue(2 * p + 2, ssem0)
            return c

        lax.fori_loop(0, NPAIR, pair, 0)
        s_wait(ssem1)
        plsc.subcore_barrier()
        pltpu.sync_copy(acc_sh.at[pl.ds(r0, ROWS_PT)],
                        out_hbm.at[cid, pl.ds(r0, ROWS_PT)])

    return k(col3, zeros128, ones128)


def _full(shape):
    return pl.BlockSpec(shape, lambda i: tuple(0 for _ in shape))


def _rows(shape):
    return pl.BlockSpec(shape, lambda i: (i,) + tuple(0 for _ in shape[1:]))


def _dot(a, b):
    return jax.lax.dot_general(a, b, (((1,), (0,)), ((), ())),
                               preferred_element_type=F32)


def _tc_node_pre(x, batch2d, cond, we, wl1):
    (w1, b1, w2, b2, wc1, bc1, wc2, bc2) = we
    (wea, web, wed, b1e, wma, b1m) = wl1

    def body(x_ref, bt_ref, cond_ref, w1_r, b1_r, w2_r, b2_r,
             wc1_r, bc1_r, wc2_r, bc2_r,
             wea_r, web_r, wed_r, b1e_r, wma_r, b1m_r,
             xh_o, ub_o, pr_o, q_o):
        u = _dot(jax.nn.relu(_dot(cond_ref[...], wc1_r[...]) + bc1_r[...]),
                 wc2_r[...]) + bc2_r[...]
        iota = lax.broadcasted_iota(jnp.int32, (NB, 16), 1)
        oh = (bt_ref[...] == iota).astype(F32)
        ub = _dot(oh, u)
        xh = _dot(jax.nn.relu(_dot(x_ref[...], w1_r[...]) + b1_r[...]),
                  w2_r[...]) + b2_r[...]
        p = _dot(xh, wea_r[...]) + _dot(ub, wed_r[...]) + b1e_r[...]
        r = _dot(xh, wma_r[...]) + b1m_r[...]
        xh_o[...] = xh
        ub_o[...] = ub
        pr_o[...] = jnp.concatenate([p, r], axis=1)
        q_o[...] = _dot(xh, web_r[...])

    return pl.pallas_call(
        body,
        grid=(N_NB,),
        in_specs=[_rows((NB, 128)), _rows((NB, 1)), _full((16, 16)),
                  _full((128, 64)), _full((1, 64)), _full((64, 64)), _full((1, 64)),
                  _full((16, 64)), _full((1, 64)), _full((64, 64)), _full((1, 64)),
                  _full((64, 64)), _full((64, 64)), _full((64, 64)), _full((1, 64)),
                  _full((64, 64)), _full((1, 64))],
        out_specs=[_rows((NB, 64)), _rows((NB, 64)),
                   _rows((NB, 128)), _rows((NB, 64))],
        out_shape=[jax.ShapeDtypeStruct((NP, 64), F32),
                   jax.ShapeDtypeStruct((NP, 64), F32),
                   jax.ShapeDtypeStruct((NP, 128), F32),
                   jax.ShapeDtypeStruct((NP, 64), F32)],
    )(x, batch2d, cond, w1, b1, w2, b2, wc1, bc1, wc2, bc2,
      wea, web, wed, b1e, wma, b1m)


def _tc_edge_enc(ea, w1, b1, w2, b2):
    def body(ea_ref, w1_r, b1_r, w2_r, b2_r, out_ref):
        h = jax.nn.relu(_dot(ea_ref[...], w1_r[...]) + b1_r[...])
        out_ref[...] = _dot(h, w2_r[...]) + b2_r[...]

    return pl.pallas_call(
        body,
        grid=(ea.shape[0] // EB,),
        in_specs=[_rows((EB, 16)), _full((16, 64)), _full((1, 64)),
                  _full((64, 64)), _full((1, 64))],
        out_specs=_rows((EB, 64)),
        out_shape=jax.ShapeDtypeStruct((ea.shape[0], 64), F32),
    )(ea, w1, b1, w2, b2)


def _tc_edge(grow, eh, wec, w2e, b2e, wmb, w2m, b2m):
    eh_w = eh.shape[1]

    def body(grow_ref, eh_ref, wec_r, w2e_r, b2e_r,
             wmb_r, w2m_r, b2m_r, em_o):
        g = grow_ref[...]
        ehv = eh_ref[...][:, :64]
        h1 = jax.nn.relu(g[:, :64] + _dot(ehv, wec_r[...]))
        ehn = _dot(h1, w2e_r[...]) + b2e_r[...]
        h2 = jax.nn.relu(g[:, 64:] + _dot(ehn, wmb_r[...]))
        m = _dot(h2, w2m_r[...]) + b2m_r[...]
        em_o[...] = jnp.concatenate([ehn, m], axis=1)

    return pl.pallas_call(
        body,
        grid=(grow.shape[0] // EB,),
        in_specs=[_rows((EB, 128)),
                  _rows((EB, eh_w)),
                  _full((64, 64)), _full((64, 64)), _full((1, 64)),
                  _full((64, 64)), _full((64, 64)), _full((1, 64))],
        out_specs=_rows((EB, 128)),
        out_shape=jax.ShapeDtypeStruct((grow.shape[0], 128), F32),
    )(grow, eh, wec, w2e, b2e, wmb, w2m, b2m)


def _tc_node(xh, s, cnt, ub, wn, wnext, pr_old, q_old):
    (wna, wnb, wnc, b1n, w2n, b2n) = wn
    (wea, web, wed, b1e, wma, b1m) = wnext

    def body(xh_ref, s0_ref, s1_ref, c0_ref, c1_ref, ub_ref,
             wna_r, wnb_r, wnc_r, b1n_r, w2n_r, b2n_r,
             wea_r, web_r, wed_r, b1e_r, wma_r, b1m_r,
             pr_old_ref, q_old_ref,
             xh_o, pr_o, q_o):
        xh0 = xh_ref[...]
        ub = ub_ref[...]
        cnt = (c0_ref[...] + c1_ref[...])[:, 0:1]
        agg = (s0_ref[...][:, 64:] + s1_ref[...][:, 64:]) / jnp.maximum(cnt, 1.0)
        h = jax.nn.relu(_dot(xh0, wna_r[...]) + _dot(agg, wnb_r[...])
                        + _dot(ub, wnc_r[...]) + b1n_r[...])
        xh = _dot(h, w2n_r[...]) + b2n_r[...] + xh0
        p = _dot(xh, wea_r[...]) + _dot(ub, wed_r[...]) + b1e_r[...]
        r = _dot(xh, wma_r[...]) + b1m_r[...]
        xh_o[...] = xh
        pr_o[...] = jnp.concatenate([p, r], axis=1)
        q_o[...] = _dot(xh, web_r[...])

    s_spec = _rows((NB, 128))
    c_spec = _rows((NB, 128))
    return pl.pallas_call(
        body,
        grid=(N_NB,),
        in_specs=[_rows((NB, 64)), s_spec, s_spec,
                  c_spec, c_spec, _rows((NB, 64)),
                  _full((64, 64)), _full((64, 64)), _full((64, 64)),
                  _full((1, 64)), _full((64, 64)), _full((1, 64)),
                  _full((64, 64)), _full((64, 64)), _full((64, 64)),
                  _full((1, 64)), _full((64, 64)), _full((1, 64)),
                  _rows((NB, 128)), _rows((NB, 64))],
        out_specs=[_rows((NB, 64)), _rows((NB, 128)), _rows((NB, 64))],
        out_shape=[jax.ShapeDtypeStruct((NP, 64), F32),
                   jax.ShapeDtypeStruct((NP, 128), F32),
                   jax.ShapeDtypeStruct((NP, 64), F32)],
        input_output_aliases={18: 1, 19: 2},
    )(xh, s[0], s[1], cnt[0], cnt[1], ub,
      wna, wnb, wnc, b1n, w2n, b2n, wea, web, wed, b1e, wma, b1m,
      pr_old, q_old)


def _tc_node_last(xh, s, cnt, ub, wn, wd):
    (wna, wnb, wnc, b1n, w2n, b2n) = wn
    (wd1, bd1, wd2, bd2) = wd

    def body(xh_ref, s0_ref, s1_ref, c0_ref, c1_ref, ub_ref,
             wna_r, wnb_r, wnc_r, b1n_r, w2n_r, b2n_r,
             wd1_r, bd1_r, wd2_r, bd2_r, out_o):
        xh0 = xh_ref[...]
        ub = ub_ref[...]
        cnt = (c0_ref[...] + c1_ref[...])[:, 0:1]
        agg = (s0_ref[...][:, 64:] + s1_ref[...][:, 64:]) / jnp.maximum(cnt, 1.0)
        h = jax.nn.relu(_dot(xh0, wna_r[...]) + _dot(agg, wnb_r[...])
                        + _dot(ub, wnc_r[...]) + b1n_r[...])
        xh = _dot(h, w2n_r[...]) + b2n_r[...] + xh0
        hd = jax.nn.relu(_dot(xh, wd1_r[...]) + bd1_r[...])
        out_o[...] = _dot(hd, wd2_r[...]) + bd2_r[...]

    s_spec = _rows((NB, 128))
    c_spec = _rows((NB, 128))
    return pl.pallas_call(
        body,
        grid=(N_NB,),
        in_specs=[_rows((NB, 64)), s_spec, s_spec,
                  c_spec, c_spec, _rows((NB, 64)),
                  _full((64, 64)), _full((64, 64)), _full((64, 64)),
                  _full((1, 64)), _full((64, 64)), _full((1, 64)),
                  _full((64, 64)), _full((1, 64)), _full((64, 128)), _full((1, 128))],
        out_specs=_rows((NB, 128)),
        out_shape=jax.ShapeDtypeStruct((NP, 128), F32),
    )(xh, s[0], s[1], cnt[0], cnt[1], ub,
      wna, wnb, wnc, b1n, w2n, b2n, wd1, bd1, wd2, bd2)


def _row(b):
    return b.reshape(1, -1)


def _split_layer(lp):
    (w1e, b1e), _ = lp["edge_mlp"]
    (w1m, b1m), _ = lp["node_mlp1"]
    return (w1e[0:64], w1e[64:128], w1e[192:256], _row(b1e),
            w1m[0:64], _row(b1m))


def kernel(x, edge_index, edge_attr, conditions, batch, params):
    x = jnp.pad(x, ((0, NP - N_R), (0, 0)))
    batch2d = jnp.pad(batch.astype(jnp.int32), (0, NP - N_R)).reshape(NP, 1)
    row_p = jnp.pad(edge_index[0].astype(jnp.int32), (0, EP - E_R))
    col_p = jnp.pad(edge_index[1].astype(jnp.int32), (0, EP - E_R),
                    constant_values=DUMMY)
    col3 = col_p.reshape(NW, N_CHUNK, CH)
    ns = 2
    eph = EP // ns
    nch = N_CHUNK // ns
    row3_h = [row_p[h * eph:(h + 1) * eph].reshape(NW, nch, CH)
              for h in range(ns)]
    col3_h = [col_p[h * eph:(h + 1) * eph].reshape(NW, nch, CH)
              for h in range(ns)]
    ea = jnp.pad(edge_attr, ((0, EP - E_R), (0, 0)))

    zeros128 = jnp.zeros((NP, 128), F32)
    ones128 = jnp.ones((CH, 128), F32)

    (ne1, ne2) = params["node_enc"]
    (ee1, ee2) = params["edge_enc"]
    (ce1, ce2) = params["cond_enc"]
    enc_w = (ne1[0], _row(ne1[1]), ne2[0], _row(ne2[1]),
             ce1[0], _row(ce1[1]), ce2[0], _row(ce2[1]))

    layers = params["layers"]
    xh, ub, pr, q = _tc_node_pre(x, batch2d, conditions, enc_w,
                                 _split_layer(layers[0]))
    eh_h = [_tc_edge_enc(ea[h * eph:(h + 1) * eph], ee1[0], _row(ee1[1]),
                         ee2[0], _row(ee2[1])) for h in range(ns)]
    cnt = _sc_count(col3, zeros128, ones128)

    for li, lp in enumerate(layers):
        (w1e, _), (w2e, b2e) = lp["edge_mlp"]
        (w1m, _), (w2m, b2m) = lp["node_mlp1"]
        grow_h = [_sc_gather(pr, q, row3_h[h], col3_h[h]) for h in range(ns)]
        eh_h = [_tc_edge(grow_h[h], eh_h[h], w1e[128:192], w2e, _row(b2e),
                         w1m[64:128], w2m, _row(b2m)) for h in range(ns)]
        s = zeros128
        for h in range(ns):
            s = _sc_scatter(eh_h[h], col3_h[h], s)
        (w1n, b1n), (w2n, b2n) = lp["node_mlp2"]
        wn = (w1n[0:64], w1n[64:128], w1n[128:192], _row(b1n), w2n, _row(b2n))
        if li + 1 < len(layers):
            xh, pr, q = _tc_node(xh, s, cnt, ub, wn,
                                 _split_layer(layers[li + 1]), pr, q)
        else:
            (nd1, nd2) = params["node_dec"]
            out = _tc_node_last(xh, s, cnt, ub, wn,
                                (nd1[0], _row(nd1[1]), nd2[0], _row(nd2[1])))
    return out[:N_R]
